# baseline (speedup 1.0000x reference)
import functools

import jax
import jax.numpy as jnp
from jax import lax
from jax.experimental import pallas as pl
from jax.experimental.pallas import tpu as pltpu
from jax.experimental.pallas import tpu_sc as plsc

N_SHOT = 16
N_WAY = 32
N_DIM = 256
N_MEM = 8192
N_SUP = N_SHOT * N_WAY
N_CAND = N_MEM + N_SHOT
N_PAD = 8320
NEG = -3.0e38
EPS = 1e-12
TOPK = 8
L = 16
N_CHUNK = N_PAD // L


def _sim_body(shat_ref, mhat_ref, out_ref):
    shat = shat_ref[...]
    mhat = mhat_ref[...]
    cosm = lax.dot_general(shat, mhat, (((1,), (1,)), ((), ())),
                           preferred_element_type=jnp.float32)
    out_ref[:, 0:N_MEM] = jnp.mean(
        cosm.reshape(N_WAY, N_SHOT, N_MEM), axis=1)

    coss = lax.dot_general(shat, shat, (((1,), (1,)), ((), ())),
                           preferred_element_type=jnp.float32)
    r = coss.reshape(N_WAY, N_SHOT, N_SUP)
    colw = lax.broadcasted_iota(jnp.int32, r.shape, 2) // N_SHOT
    roww = lax.broadcasted_iota(jnp.int32, r.shape, 0)
    blk = jnp.mean(jnp.where(colw == roww, r, 0.0), axis=1)
    b3 = blk.reshape(N_WAY, N_WAY, N_SHOT)
    kk = lax.broadcasted_iota(jnp.int32, b3.shape, 1)
    ww = lax.broadcasted_iota(jnp.int32, b3.shape, 0)
    sup_sim = jnp.sum(jnp.where(kk == ww, b3, 0.0), axis=1)

    tail = jnp.concatenate(
        [sup_sim, jnp.full((N_WAY, N_PAD - N_CAND), NEG, jnp.float32)], axis=1)
    out_ref[:, N_MEM:N_PAD] = tail


_sim_tc = pl.pallas_call(
    _sim_body,
    out_shape=jax.ShapeDtypeStruct((N_WAY, N_PAD), jnp.float32),
)


def _sc_body(sim_hbm, mem_hbm, sup_hbm, out_hbm,
             sim_v, midx_v, sidx_v, mrows_v, srows_v, acc_v, sem):
    wid = lax.axis_index("s") * 2 + lax.axis_index("c")
    iota = lax.iota(jnp.int32, L)
    lane_lt8 = iota < TOPK

    pltpu.sync_copy(sim_hbm.at[wid], sim_v)

    def _take16(x, idx):
        dn = lax.GatherDimensionNumbers(
            offset_dims=(), collapsed_slice_dims=(0,), start_index_map=(0,))
        return lax.gather(x, idx[:, None], dn, slice_sizes=(1,),
                          mode=lax.GatherScatterMode.PROMISE_IN_BOUNDS)

    def _bcast(x, lane):
        return _take16(x, jnp.full((L,), lane, jnp.int32))

    def _insert(rs, qs, t, ti):
        for k in range(TOPK):
            m = t > rs[k]
            rs[k], t = jnp.where(m, t, rs[k]), jnp.where(m, rs[k], t)
            qs[k], ti = jnp.where(m, ti, qs[k]), jnp.where(m, qs[k], ti)

    tv0 = jnp.full((L,), NEG, jnp.float32)
    ti0 = jnp.zeros((L,), jnp.int32)

    def scan_body(i, carry):
        ra, qa, rb, qb = (list(x) for x in carry)
        base = i * (2 * L)
        _insert(ra, qa, sim_v[pl.ds(base, L)], iota + base)
        _insert(rb, qb, sim_v[pl.ds(base + L, L)], iota + (base + L))
        return tuple(ra), tuple(qa), tuple(rb), tuple(qb)

    init = ((tv0,) * TOPK, (ti0,) * TOPK, (tv0,) * TOPK, (ti0,) * TOPK)
    ra, qa, rb, qb = plsc.parallel_loop(
        0, N_CHUNK // 2, 1, unroll=4, carry=init)(scan_body)

    top_v, top_i = tv0, ti0
    for vv, ii in list(zip(ra, qa)) + list(zip(rb, qb)):
        sv, si = plsc.sort_key_val(vv, ii, descending=True)
        cv = jnp.where(lane_lt8, top_v, lax.rev(sv, (0,)))
        ci = jnp.where(lane_lt8, top_i, lax.rev(si, (0,)))
        top_v, top_i = plsc.sort_key_val(cv, ci, descending=True)

    w_all = jnp.where(lane_lt8, top_v, 0.0)
    denom = w_all
    for off in (8, 4, 2, 1):
        denom = denom + _take16(denom, iota ^ off)

    is_mem = lane_lt8 & (top_i < N_MEM)
    is_sup = lane_lt8 & (top_i >= N_MEM)
    w_mem = jnp.where(is_mem, w_all, 0.0)
    w_sup = jnp.where(is_sup, w_all, 0.0)
    midx_v[...] = jnp.where(is_mem, top_i, 0)
    sidx_v[...] = jnp.where(is_sup, top_i - N_MEM + wid * N_SHOT, 0)

    pltpu.async_copy(mem_hbm.at[midx_v], mrows_v, sem).wait()
    pltpu.async_copy(sup_hbm.at[sidx_v], srows_v, sem).wait()

    wm = [_bcast(w_mem, r) for r in range(L)]
    ws = [_bcast(w_sup, r) for r in range(L)]
    for d in range(N_DIM // L):
        acc = jnp.zeros((L,), jnp.float32)
        for r in range(L):
            acc = acc + wm[r] * mrows_v[r, pl.ds(d * L, L)]
            acc = acc + ws[r] * srows_v[r, pl.ds(d * L, L)]
        acc_v[pl.ds(d * L, L)] = acc / denom

    pltpu.sync_copy(acc_v, out_hbm.at[wid])


@functools.cache
def _make_sc_topk():
    return functools.partial(
        pl.kernel,
        out_type=jax.ShapeDtypeStruct((N_WAY, N_DIM), jnp.float32),
        mesh=plsc.VectorSubcoreMesh(core_axis_name="c", subcore_axis_name="s"),
        compiler_params=pltpu.CompilerParams(needs_layout_passes=False),
        scratch_types=[
            pltpu.VMEM((N_PAD,), jnp.float32),
            pltpu.VMEM((L,), jnp.int32),
            pltpu.VMEM((L,), jnp.int32),
            pltpu.VMEM((L, N_DIM), jnp.float32),
            pltpu.VMEM((L, N_DIM), jnp.float32),
            pltpu.VMEM((N_DIM,), jnp.float32),
            pltpu.SemaphoreType.DMA,
        ],
    )(_sc_body)


def kernel(support, memory):
    sup_t = jnp.transpose(support, (0, 2, 1, 3))
    sn = jnp.linalg.norm(sup_t, axis=-1, keepdims=True)
    shat = (sup_t / jnp.maximum(sn, EPS))[0].reshape(N_SUP, N_DIM)
    mn = jnp.linalg.norm(memory, axis=-1, keepdims=True)
    mhat = memory / jnp.maximum(mn, EPS)

    sim = _sim_tc(shat, mhat)
    sup_flat = sup_t.reshape(N_SUP, N_DIM)
    proto = _make_sc_topk()(sim, memory, sup_flat)
    return proto.reshape(1, N_WAY, N_DIM)


# speedup vs baseline: 1.0726x; 1.0726x over previous
import functools

import jax
import jax.numpy as jnp
from jax import lax
from jax.experimental import pallas as pl
from jax.experimental.pallas import tpu as pltpu
from jax.experimental.pallas import tpu_sc as plsc

N_SHOT = 16
N_WAY = 32
N_DIM = 256
N_MEM = 8192
N_SUP = N_SHOT * N_WAY
N_CAND = N_MEM + N_SHOT
N_PAD = 8320
NEG = -3.0e38
EPS = 1e-12
TOPK = 8
L = 16
N_CHUNK = N_PAD // L


def _sim_body(shat_ref, mhat_ref, out_ref):
    shat = shat_ref[...]
    mhat = mhat_ref[...]
    cosm = lax.dot_general(shat, mhat, (((1,), (1,)), ((), ())),
                           preferred_element_type=jnp.float32)
    out_ref[:, 0:N_MEM] = jnp.mean(
        cosm.reshape(N_WAY, N_SHOT, N_MEM), axis=1)

    coss = lax.dot_general(shat, shat, (((1,), (1,)), ((), ())),
                           preferred_element_type=jnp.float32)
    r = coss.reshape(N_WAY, N_SHOT, N_SUP)
    colw = lax.broadcasted_iota(jnp.int32, r.shape, 2) // N_SHOT
    roww = lax.broadcasted_iota(jnp.int32, r.shape, 0)
    blk = jnp.mean(jnp.where(colw == roww, r, 0.0), axis=1)
    b3 = blk.reshape(N_WAY, N_WAY, N_SHOT)
    kk = lax.broadcasted_iota(jnp.int32, b3.shape, 1)
    ww = lax.broadcasted_iota(jnp.int32, b3.shape, 0)
    sup_sim = jnp.sum(jnp.where(kk == ww, b3, 0.0), axis=1)

    tail = jnp.concatenate(
        [sup_sim, jnp.full((N_WAY, N_PAD - N_CAND), NEG, jnp.float32)], axis=1)
    out_ref[:, N_MEM:N_PAD] = tail


_sim_tc = pl.pallas_call(
    _sim_body,
    out_shape=jax.ShapeDtypeStruct((N_WAY, N_PAD), jnp.float32),
)


def _sc_body(sim_hbm, mem_hbm, sup_hbm, out_hbm,
             sim_v, midx_v, sidx_v, mrows_v, srows_v, acc_v, sem, sem2):
    wid = lax.axis_index("s") * 2 + lax.axis_index("c")
    iota = lax.iota(jnp.int32, L)
    lane_lt8 = iota < TOPK

    pltpu.sync_copy(sim_hbm.at[wid], sim_v)

    def _take16(x, idx):
        dn = lax.GatherDimensionNumbers(
            offset_dims=(), collapsed_slice_dims=(0,), start_index_map=(0,))
        return lax.gather(x, idx[:, None], dn, slice_sizes=(1,),
                          mode=lax.GatherScatterMode.PROMISE_IN_BOUNDS)

    def _bcast(x, lane):
        return _take16(x, jnp.full((L,), lane, jnp.int32))

    def _insert(rs, qs, t, ti):
        for k in range(TOPK):
            m = t > rs[k]
            rs[k], t = jnp.where(m, t, rs[k]), jnp.where(m, rs[k], t)
            qs[k], ti = jnp.where(m, ti, qs[k]), jnp.where(m, qs[k], ti)

    tv0 = jnp.full((L,), NEG, jnp.float32)
    ti0 = jnp.zeros((L,), jnp.int32)

    def scan_body(i, carry):
        ra, qa, rb, qb = (list(x) for x in carry)
        base = i * (2 * L)
        _insert(ra, qa, sim_v[pl.ds(base, L)], iota + base)
        _insert(rb, qb, sim_v[pl.ds(base + L, L)], iota + (base + L))
        return tuple(ra), tuple(qa), tuple(rb), tuple(qb)

    init = ((tv0,) * TOPK, (ti0,) * TOPK, (tv0,) * TOPK, (ti0,) * TOPK)
    ra, qa, rb, qb = plsc.parallel_loop(
        0, N_CHUNK // 2, 1, unroll=4, carry=init)(scan_body)

    top_v, top_i = tv0, ti0
    for vv, ii in list(zip(ra, qa)) + list(zip(rb, qb)):
        sv, si = plsc.sort_key_val(vv, ii, descending=True)
        cv = jnp.where(lane_lt8, top_v, lax.rev(sv, (0,)))
        ci = jnp.where(lane_lt8, top_i, lax.rev(si, (0,)))
        top_v, top_i = plsc.sort_key_val(cv, ci, descending=True)

    w_all = jnp.where(lane_lt8, top_v, 0.0)
    denom = w_all
    for off in (8, 4, 2, 1):
        denom = denom + _take16(denom, iota ^ off)

    is_mem = lane_lt8 & (top_i < N_MEM)
    is_sup = lane_lt8 & (top_i >= N_MEM)
    w_mem = jnp.where(is_mem, w_all, 0.0)
    w_sup = jnp.where(is_sup, w_all, 0.0)
    midx_v[...] = jnp.where(is_mem, top_i, 0)
    sidx_v[...] = jnp.where(is_sup, top_i - N_MEM + wid * N_SHOT, 0)

    cp_m = pltpu.async_copy(mem_hbm.at[midx_v], mrows_v, sem)
    cp_s = pltpu.async_copy(sup_hbm.at[sidx_v], srows_v, sem2)
    cp_m.wait()
    cp_s.wait()

    wm = [_bcast(w_mem, r) for r in range(L)]
    ws = [_bcast(w_sup, r) for r in range(L)]
    for d in range(N_DIM // L):
        acc = jnp.zeros((L,), jnp.float32)
        for r in range(L):
            acc = acc + wm[r] * mrows_v[r, pl.ds(d * L, L)]
            acc = acc + ws[r] * srows_v[r, pl.ds(d * L, L)]
        acc_v[pl.ds(d * L, L)] = acc / denom

    pltpu.sync_copy(acc_v, out_hbm.at[wid])


@functools.cache
def _make_sc_topk():
    return functools.partial(
        pl.kernel,
        out_type=jax.ShapeDtypeStruct((N_WAY, N_DIM), jnp.float32),
        mesh=plsc.VectorSubcoreMesh(core_axis_name="c", subcore_axis_name="s"),
        compiler_params=pltpu.CompilerParams(needs_layout_passes=False),
        scratch_types=[
            pltpu.VMEM((N_PAD,), jnp.float32),
            pltpu.VMEM((L,), jnp.int32),
            pltpu.VMEM((L,), jnp.int32),
            pltpu.VMEM((L, N_DIM), jnp.float32),
            pltpu.VMEM((L, N_DIM), jnp.float32),
            pltpu.VMEM((N_DIM,), jnp.float32),
            pltpu.SemaphoreType.DMA,
            pltpu.SemaphoreType.DMA,
        ],
    )(_sc_body)


def kernel(support, memory):
    sup_t = jnp.transpose(support, (0, 2, 1, 3))
    sn = jnp.linalg.norm(sup_t, axis=-1, keepdims=True)
    shat = (sup_t / jnp.maximum(sn, EPS))[0].reshape(N_SUP, N_DIM)
    mn = jnp.linalg.norm(memory, axis=-1, keepdims=True)
    mhat = memory / jnp.maximum(mn, EPS)

    sim = _sim_tc(shat, mhat)
    sup_flat = sup_t.reshape(N_SUP, N_DIM)
    proto = _make_sc_topk()(sim, memory, sup_flat)
    return proto.reshape(1, N_WAY, N_DIM)


# speedup vs baseline: 1.2707x; 1.1847x over previous
import functools

import jax
import jax.numpy as jnp
from jax import lax
from jax.experimental import pallas as pl
from jax.experimental.pallas import tpu as pltpu
from jax.experimental.pallas import tpu_sc as plsc

N_SHOT = 16
N_WAY = 32
N_DIM = 256
N_MEM = 8192
N_SUP = N_SHOT * N_WAY
N_CAND = N_MEM + N_SHOT
N_PAD = 8320
NEG = -3.0e38
EPS = 1e-12
TOPK = 8
L = 16
N_CHUNK = N_PAD // L


def _sim_body(shat_ref, mhat_ref, out_ref):
    shat = shat_ref[...]
    mhat = mhat_ref[...]
    cosm = lax.dot_general(shat, mhat, (((1,), (1,)), ((), ())),
                           preferred_element_type=jnp.float32)
    out_ref[:, 0:N_MEM] = jnp.mean(
        cosm.reshape(N_WAY, N_SHOT, N_MEM), axis=1)

    coss = lax.dot_general(shat, shat, (((1,), (1,)), ((), ())),
                           preferred_element_type=jnp.float32)
    r = coss.reshape(N_WAY, N_SHOT, N_SUP)
    colw = lax.broadcasted_iota(jnp.int32, r.shape, 2) // N_SHOT
    roww = lax.broadcasted_iota(jnp.int32, r.shape, 0)
    blk = jnp.mean(jnp.where(colw == roww, r, 0.0), axis=1)
    b3 = blk.reshape(N_WAY, N_WAY, N_SHOT)
    kk = lax.broadcasted_iota(jnp.int32, b3.shape, 1)
    ww = lax.broadcasted_iota(jnp.int32, b3.shape, 0)
    sup_sim = jnp.sum(jnp.where(kk == ww, b3, 0.0), axis=1)

    tail = jnp.concatenate(
        [sup_sim, jnp.full((N_WAY, N_PAD - N_CAND), NEG, jnp.float32)], axis=1)
    out_ref[:, N_MEM:N_PAD] = tail


_sim_tc = pl.pallas_call(
    _sim_body,
    out_shape=jax.ShapeDtypeStruct((N_WAY, N_PAD), jnp.float32),
)


def _sc_body(sim_hbm, mem_hbm, sup_hbm, out_hbm,
             sim_v, mrows_v, srows_v, acc_v, sem, sem2):
    wid = lax.axis_index("s") * 2 + lax.axis_index("c")
    iota = lax.iota(jnp.int32, L)
    lane_lt8 = iota < TOPK

    pltpu.sync_copy(sim_hbm.at[wid], sim_v)

    def _take16(x, idx):
        dn = lax.GatherDimensionNumbers(
            offset_dims=(), collapsed_slice_dims=(0,), start_index_map=(0,))
        return lax.gather(x, idx[:, None], dn, slice_sizes=(1,),
                          mode=lax.GatherScatterMode.PROMISE_IN_BOUNDS)

    def _bcast(x, lane):
        return _take16(x, jnp.full((L,), lane, jnp.int32))

    def _insert(rs, qs, t, ti):
        for k in range(TOPK):
            m = t > rs[k]
            rs[k], t = jnp.where(m, t, rs[k]), jnp.where(m, rs[k], t)
            qs[k], ti = jnp.where(m, ti, qs[k]), jnp.where(m, qs[k], ti)

    tv0 = jnp.full((L,), NEG, jnp.float32)
    ti0 = jnp.zeros((L,), jnp.int32)

    def scan_body(i, carry):
        ra, qa, rb, qb = (list(x) for x in carry)
        base = i * (2 * L)
        _insert(ra, qa, sim_v[pl.ds(base, L)], iota + base)
        _insert(rb, qb, sim_v[pl.ds(base + L, L)], iota + (base + L))
        return tuple(ra), tuple(qa), tuple(rb), tuple(qb)

    init = ((tv0,) * TOPK, (ti0,) * TOPK, (tv0,) * TOPK, (ti0,) * TOPK)
    ra, qa, rb, qb = plsc.parallel_loop(
        0, N_CHUNK // 2, 1, unroll=4, carry=init)(scan_body)

    top_v, top_i = tv0, ti0
    for vv, ii in list(zip(ra, qa)) + list(zip(rb, qb)):
        sv, si = plsc.sort_key_val(vv, ii, descending=True)
        cv = jnp.where(lane_lt8, top_v, lax.rev(sv, (0,)))
        ci = jnp.where(lane_lt8, top_i, lax.rev(si, (0,)))
        top_v, top_i = plsc.sort_key_val(cv, ci, descending=True)

    w_all = jnp.where(lane_lt8, top_v, 0.0)
    denom = w_all
    for off in (8, 4, 2, 1):
        denom = denom + _take16(denom, iota ^ off)

    is_mem = lane_lt8 & (top_i < N_MEM)
    is_sup = lane_lt8 & (top_i >= N_MEM)
    w_mem = jnp.where(is_mem, w_all, 0.0)
    w_sup = jnp.where(is_sup, w_all, 0.0)
    midx = jnp.where(is_mem, top_i, 0)
    sidx = jnp.where(is_sup, top_i - N_MEM + wid * N_SHOT, 0)

    cps = []
    for r in range(TOPK):
        mi = jnp.squeeze(lax.slice(midx, (r,), (r + 1,)))
        si = jnp.squeeze(lax.slice(sidx, (r,), (r + 1,)))
        cps.append(pltpu.async_copy(mem_hbm.at[mi], mrows_v.at[r], sem))
        cps.append(pltpu.async_copy(sup_hbm.at[si], srows_v.at[r], sem2))
    for cp in cps:
        cp.wait()

    wm = [_bcast(w_mem, r) for r in range(TOPK)]
    ws = [_bcast(w_sup, r) for r in range(TOPK)]
    for d in range(N_DIM // L):
        acc = jnp.zeros((L,), jnp.float32)
        for r in range(TOPK):
            acc = acc + wm[r] * mrows_v[r, pl.ds(d * L, L)]
            acc = acc + ws[r] * srows_v[r, pl.ds(d * L, L)]
        acc_v[pl.ds(d * L, L)] = acc / denom

    pltpu.sync_copy(acc_v, out_hbm.at[wid])


@functools.cache
def _make_sc_topk():
    return functools.partial(
        pl.kernel,
        out_type=jax.ShapeDtypeStruct((N_WAY, N_DIM), jnp.float32),
        mesh=plsc.VectorSubcoreMesh(core_axis_name="c", subcore_axis_name="s"),
        compiler_params=pltpu.CompilerParams(needs_layout_passes=False),
        scratch_types=[
            pltpu.VMEM((N_PAD,), jnp.float32),
            pltpu.VMEM((TOPK, N_DIM), jnp.float32),
            pltpu.VMEM((TOPK, N_DIM), jnp.float32),
            pltpu.VMEM((N_DIM,), jnp.float32),
            pltpu.SemaphoreType.DMA,
            pltpu.SemaphoreType.DMA,
        ],
    )(_sc_body)


def kernel(support, memory):
    sup_t = jnp.transpose(support, (0, 2, 1, 3))
    sn = jnp.linalg.norm(sup_t, axis=-1, keepdims=True)
    shat = (sup_t / jnp.maximum(sn, EPS))[0].reshape(N_SUP, N_DIM)
    mn = jnp.linalg.norm(memory, axis=-1, keepdims=True)
    mhat = memory / jnp.maximum(mn, EPS)

    sim = _sim_tc(shat, mhat)
    sup_flat = sup_t.reshape(N_SUP, N_DIM)
    proto = _make_sc_topk()(sim, memory, sup_flat)
    return proto.reshape(1, N_WAY, N_DIM)


# speedup vs baseline: 1.2719x; 1.0009x over previous
import functools

import jax
import jax.numpy as jnp
from jax import lax
from jax.experimental import pallas as pl
from jax.experimental.pallas import tpu as pltpu
from jax.experimental.pallas import tpu_sc as plsc

N_SHOT = 16
N_WAY = 32
N_DIM = 256
N_MEM = 8192
N_SUP = N_SHOT * N_WAY
N_PAD = 9216
PANEL = 1024
NEG = -3.0e38
EPS = 1e-12
TOPK = 8
L = 16


def _sim_body(shat_ref, mem_ref, mn_ref, out_ref):
    p = pl.program_id(0)

    @pl.when(p < 8)
    def _mem_panel():
        mhat = mem_ref[...] / jnp.maximum(mn_ref[...], EPS)
        cosm = lax.dot_general(shat_ref[...], mhat, (((1,), (1,)), ((), ())),
                               preferred_element_type=jnp.float32)
        out_ref[...] = jnp.mean(cosm.reshape(N_WAY, N_SHOT, PANEL), axis=1)

    @pl.when(p == 8)
    def _sup_panel():
        shat = shat_ref[...]
        coss = lax.dot_general(shat, shat, (((1,), (1,)), ((), ())),
                               preferred_element_type=jnp.float32)
        r = coss.reshape(N_WAY, N_SHOT, N_SUP)
        colw = lax.broadcasted_iota(jnp.int32, r.shape, 2) // N_SHOT
        roww = lax.broadcasted_iota(jnp.int32, r.shape, 0)
        blk = jnp.mean(jnp.where(colw == roww, r, 0.0), axis=1)
        b3 = blk.reshape(N_WAY, N_WAY, N_SHOT)
        kk = lax.broadcasted_iota(jnp.int32, b3.shape, 1)
        ww = lax.broadcasted_iota(jnp.int32, b3.shape, 0)
        sup_sim = jnp.sum(jnp.where(kk == ww, b3, 0.0), axis=1)
        out_ref[...] = jnp.concatenate(
            [sup_sim, jnp.full((N_WAY, PANEL - N_SHOT), NEG, jnp.float32)],
            axis=1)


_sim_tc = pl.pallas_call(
    _sim_body,
    grid=(9,),
    in_specs=[
        pl.BlockSpec((N_SUP, N_DIM), lambda p: (0, 0)),
        pl.BlockSpec((PANEL, N_DIM), lambda p: (jnp.minimum(p, 7), 0)),
        pl.BlockSpec((PANEL, 1), lambda p: (jnp.minimum(p, 7), 0)),
    ],
    out_specs=pl.BlockSpec((N_WAY, PANEL), lambda p: (0, p)),
    out_shape=jax.ShapeDtypeStruct((N_WAY, N_PAD), jnp.float32),
)


def _sc_body(sim_hbm, mem_hbm, sup_hbm, out_hbm,
             sim_v, sup16_v, mrows_v, acc_v, sem, sem2):
    wid = lax.axis_index("s") * 2 + lax.axis_index("c")
    iota = lax.iota(jnp.int32, L)
    lane_lt8 = iota < TOPK

    cp_sup = pltpu.async_copy(
        sup_hbm.at[pl.ds(wid * N_SHOT, N_SHOT)], sup16_v, sem2)
    pltpu.sync_copy(sim_hbm.at[wid], sim_v)

    def _take16(x, idx):
        dn = lax.GatherDimensionNumbers(
            offset_dims=(), collapsed_slice_dims=(0,), start_index_map=(0,))
        return lax.gather(x, idx[:, None], dn, slice_sizes=(1,),
                          mode=lax.GatherScatterMode.PROMISE_IN_BOUNDS)

    def _bcast(x, lane):
        return _take16(x, jnp.full((L,), lane, jnp.int32))

    def _insert(rs, qs, t, ti):
        for k in range(TOPK):
            m = t > rs[k]
            rs[k], t = jnp.where(m, t, rs[k]), jnp.where(m, rs[k], t)
            qs[k], ti = jnp.where(m, ti, qs[k]), jnp.where(m, qs[k], ti)

    tv0 = jnp.full((L,), NEG, jnp.float32)
    ti0 = jnp.zeros((L,), jnp.int32)

    def scan_body(i, carry):
        ra, qa, rb, qb = (list(x) for x in carry)
        base = i * (2 * L)
        _insert(ra, qa, sim_v[pl.ds(base, L)], iota + base)
        _insert(rb, qb, sim_v[pl.ds(base + L, L)], iota + (base + L))
        return tuple(ra), tuple(qa), tuple(rb), tuple(qb)

    init = ((tv0,) * TOPK, (ti0,) * TOPK, (tv0,) * TOPK, (ti0,) * TOPK)
    ra, qa, rb, qb = plsc.parallel_loop(
        0, N_MEM // (2 * L), 1, unroll=4, carry=init)(scan_body)

    ra, qa = list(ra), list(qa)
    _insert(ra, qa, sim_v[pl.ds(N_MEM, L)], iota + N_MEM)

    top_v, top_i = tv0, ti0
    for vv, ii in list(zip(ra, qa)) + list(zip(rb, qb)):
        sv, si = plsc.sort_key_val(vv, ii, descending=True)
        cv = jnp.where(lane_lt8, top_v, lax.rev(sv, (0,)))
        ci = jnp.where(lane_lt8, top_i, lax.rev(si, (0,)))
        top_v, top_i = plsc.sort_key_val(cv, ci, descending=True)

    w_all = jnp.where(lane_lt8, top_v, 0.0)
    denom = w_all
    for off in (8, 4, 2, 1):
        denom = denom + _take16(denom, iota ^ off)

    is_mem = lane_lt8 & (top_i < N_MEM)
    is_sup = lane_lt8 & (top_i >= N_MEM)
    w_mem = jnp.where(is_mem, w_all, 0.0)
    w_sup = jnp.where(is_sup, w_all, 0.0)
    midx = jnp.where(is_mem, top_i, 0)
    sidx = jnp.where(is_sup, top_i - N_MEM, 0)

    cps = []
    for r in range(TOPK):
        mi = jnp.squeeze(lax.slice(midx, (r,), (r + 1,)))
        cps.append(pltpu.async_copy(mem_hbm.at[mi], mrows_v.at[r], sem))
    cp_sup.wait()
    for cp in cps:
        cp.wait()

    wm = [_bcast(w_mem, r) for r in range(TOPK)]
    ws = [_bcast(w_sup, r) for r in range(TOPK)]
    sj = [jnp.squeeze(lax.slice(sidx, (r,), (r + 1,))) for r in range(TOPK)]
    for d in range(N_DIM // L):
        acc = jnp.zeros((L,), jnp.float32)
        for r in range(TOPK):
            acc = acc + wm[r] * mrows_v[r, pl.ds(d * L, L)]
            acc = acc + ws[r] * sup16_v[sj[r], pl.ds(d * L, L)]
        acc_v[pl.ds(d * L, L)] = acc / denom

    pltpu.sync_copy(acc_v, out_hbm.at[wid])


@functools.cache
def _make_sc_topk():
    return functools.partial(
        pl.kernel,
        out_type=jax.ShapeDtypeStruct((N_WAY, N_DIM), jnp.float32),
        mesh=plsc.VectorSubcoreMesh(core_axis_name="c", subcore_axis_name="s"),
        compiler_params=pltpu.CompilerParams(needs_layout_passes=False),
        scratch_types=[
            pltpu.VMEM((N_PAD,), jnp.float32),
            pltpu.VMEM((N_SHOT, N_DIM), jnp.float32),
            pltpu.VMEM((TOPK, N_DIM), jnp.float32),
            pltpu.VMEM((N_DIM,), jnp.float32),
            pltpu.SemaphoreType.DMA,
            pltpu.SemaphoreType.DMA,
        ],
    )(_sc_body)


def kernel(support, memory):
    sup_t = jnp.transpose(support, (0, 2, 1, 3))
    sn = jnp.linalg.norm(sup_t, axis=-1, keepdims=True)
    shat = (sup_t / jnp.maximum(sn, EPS))[0].reshape(N_SUP, N_DIM)
    mn = jnp.linalg.norm(memory, axis=-1, keepdims=True)

    sim = _sim_tc(shat, memory, mn)
    sup_flat = sup_t.reshape(N_SUP, N_DIM)
    proto = _make_sc_topk()(sim, memory, sup_flat)
    return proto.reshape(1, N_WAY, N_DIM)
